# Initial kernel scaffold; baseline (speedup 1.0000x reference)
import jax
import jax.numpy as jnp
from jax.experimental import pallas as pl


def kernel(queue, queue_labels, queue_ptr, features, labels):
    raise NotImplementedError("write your pallas kernel here")


# speedup vs baseline: 3.8034x; 3.8034x over previous
import jax
import jax.numpy as jnp
from jax import lax
from jax.experimental import pallas as pl
from jax.experimental.pallas import tpu as pltpu

K = 32768
D = 2048
B = 4096
R = 256
NB = K // R
FP = B + 2 * R


def _body(ptr_ref, q_ref, feat_ref, qlab_ref, lab_ref, outq_ref, outl_ref):
    j = pl.program_id(0)
    p0 = ptr_ref[0]
    d = lax.rem(j * R - p0 + K, K)
    s = jnp.where(d < K - R, d, d - K)
    start = pl.multiple_of(jnp.clip(s, -R, B) + R, 8)
    p = lax.broadcasted_iota(jnp.int32, (R, 1), 0)
    i = s + p
    mask = (i >= 0) & (i < B)
    fs = feat_ref[pl.ds(start, R), :]
    outq_ref[...] = jnp.where(mask, fs, q_ref[...])
    ls = lab_ref[pl.ds(start, R), :]
    outl_ref[...] = jnp.where(mask, ls, qlab_ref[...])


def kernel(queue, queue_labels, queue_ptr, features, labels):
    ptr = jnp.asarray(queue_ptr, jnp.int32).reshape(1)
    feat_pad = jnp.pad(features, ((R, R), (0, 0)))
    lab_pad = jnp.pad(labels.astype(jnp.int32), (R, R)).reshape(FP, 1)
    qlab2 = queue_labels.astype(jnp.int32).reshape(K, 1)

    new_queue, new_lab2 = pl.pallas_call(
        _body,
        grid=(NB,),
        in_specs=[
            pl.BlockSpec(memory_space=pltpu.SMEM),
            pl.BlockSpec((R, D), lambda j: (j, 0)),
            pl.BlockSpec((FP, D), lambda j: (0, 0)),
            pl.BlockSpec((R, 1), lambda j: (j, 0)),
            pl.BlockSpec((FP, 1), lambda j: (0, 0)),
        ],
        out_specs=[
            pl.BlockSpec((R, D), lambda j: (j, 0)),
            pl.BlockSpec((R, 1), lambda j: (j, 0)),
        ],
        out_shape=[
            jax.ShapeDtypeStruct((K, D), jnp.float32),
            jax.ShapeDtypeStruct((K, 1), jnp.int32),
        ],
    )(ptr, queue, feat_pad, qlab2, lab_pad)

    new_ptr = jnp.asarray(lax.rem(jnp.asarray(queue_ptr, jnp.int32) + B, K), jnp.int32)
    return new_queue, new_lab2.reshape(K), new_ptr
